# baseline (speedup 1.0000x reference)
import functools

import jax
import jax.numpy as jnp
from jax import lax
from jax.experimental import pallas as pl
from jax.experimental.pallas import tpu as pltpu
from jax.experimental.pallas import tpu_sc as plsc

_L = 16
_D = 128
_D_IN = 260
_N = 100000
_CHUNK = 125
_BUFROWS = 136
_NW = 32
_PER_W = 4
_SC_ROWS = _NW * _PER_W * _CHUNK
_TC_ROWS = _N - _SC_ROWS
_TC_BLK = 7000
_WSTRIDE = 504


def _sc_scores_body(hv_hbm, w1a_hbm, w2d_hbm, out_hbm,
                    buf0, buf1, buf2, w1buf, w2buf, va2buf, sbuf,
                    semw, sem0, sem1, sem2):
    wid = lax.axis_index("s") * 2 + lax.axis_index("c")
    iota = lax.iota(jnp.int32, _L)
    first = wid * _PER_W

    def chunk_src(t):
        start = _TC_ROWS + (first + t) * _CHUNK
        aligned = pl.multiple_of(
            jnp.minimum((start // 8) * 8, _N - _BUFROWS), 8)
        return hv_hbm.at[pl.ds(aligned, _BUFROWS)], start - aligned

    def fire(t, buf, sem):
        src, _ = chunk_src(t)
        pltpu.async_copy(src, buf, sem)

    pltpu.async_copy(w1a_hbm, w1buf, semw)
    fire(0, buf0, sem0)
    pltpu.sync_copy(w2d_hbm, w2buf)
    pltpu.make_async_copy(w1a_hbm, w1buf, semw).wait()

    w1base = [(16 * j + iota) * _D_IN for j in range(_D // _L)]

    def va_step(k, accs):
        kv = k + iota
        kv = jnp.where(kv >= _D_IN, kv - _D_IN, kv)
        w2k = w2buf[pl.ds(k, _L)]
        return tuple(
            accs[j] + plsc.load_gather(w1buf, [w1base[j] + kv]) * w2k
            for j in range(_D // _L))

    va = lax.fori_loop(0, _D_IN, va_step,
                       tuple(jnp.zeros((_L,), jnp.float32)
                             for _ in range(_D // _L)), unroll=4)
    for j in range(_D // _L):
        va2buf[pl.ds(16 * j, _L)] = va[j]
        va2buf[pl.ds(_D + 16 * j, _L)] = va[j]

    ngroups = (_CHUNK + _L - 1) // _L

    def compute(t, buf):
        _, extra = chunk_src(t)
        rows = [jnp.minimum(16 * g + iota, _CHUNK - 1) + extra
                for g in range(ngroups)]

        def d_step(d, accs):
            m = jnp.bitwise_and(d + iota, _D - 1)
            vad = va2buf[pl.ds(d, _L)]
            return tuple(
                accs[g] + plsc.load_gather(buf, [rows[g], m]) * vad
                for g in range(ngroups))

        accs = lax.fori_loop(
            0, _D, d_step,
            tuple(jnp.zeros((_L,), jnp.float32) for _ in range(ngroups)),
            unroll=4)
        for g in range(ngroups):
            sbuf[pl.ds(t * _CHUNK + 16 * g, _L)] = accs[g]

    def wait(t, buf, sem):
        src, _ = chunk_src(t)
        pltpu.make_async_copy(src, buf, sem).wait()

    fire(1, buf1, sem1)
    fire(2, buf2, sem2)

    def triple_step(p, carry):
        t0 = 3 * p
        for i, (b, sm) in enumerate(((buf0, sem0), (buf1, sem1),
                                     (buf2, sem2))):
            t = t0 + i
            wait(t, b, sm)
            compute(t, b)

            @pl.when(t + 3 < _PER_W)
            def _():
                fire(t + 3, b, sm)
        return carry

    lax.fori_loop(0, _PER_W // 3, triple_step, 0)
    wait(_PER_W - 1, buf0, sem0)
    compute(_PER_W - 1, buf0)
    pltpu.sync_copy(sbuf, out_hbm.at[pl.ds(wid * _WSTRIDE, _WSTRIDE)])


def _sc_scores(hv, w1af, w2d):
    mesh = plsc.VectorSubcoreMesh(core_axis_name="c", subcore_axis_name="s")
    f = functools.partial(
        pl.kernel, mesh=mesh,
        out_type=jax.ShapeDtypeStruct((_NW * _WSTRIDE,), jnp.float32),
        scratch_types=[
            pltpu.VMEM((_BUFROWS, _D), jnp.float32),
            pltpu.VMEM((_BUFROWS, _D), jnp.float32),
            pltpu.VMEM((_BUFROWS, _D), jnp.float32),
            pltpu.VMEM((_D * _D_IN,), jnp.float32),
            pltpu.VMEM((2 * _D_IN,), jnp.float32),
            pltpu.VMEM((2 * _D,), jnp.float32),
            pltpu.VMEM((_WSTRIDE,), jnp.float32),
            pltpu.SemaphoreType.DMA,
            pltpu.SemaphoreType.DMA,
            pltpu.SemaphoreType.DMA,
            pltpu.SemaphoreType.DMA,
        ],
        compiler_params=pltpu.CompilerParams(needs_layout_passes=False),
    )(_sc_scores_body)
    return f(hv, w1af, w2d)


def _tc_scores_body(hv_ref, w1_ref, w2r_ref, out_ref, va_ref):
    d = hv_ref.shape[1]

    @pl.when(pl.program_id(0) == 0)
    def _():
        va_ref[...] = lax.dot_general(
            w2r_ref[...], w1_ref[0:d, :], (((1,), (1,)), ((), ())),
            preferred_element_type=jnp.float32)

    s_blk = lax.dot_general(
        va_ref[...], hv_ref[...], (((1,), (1,)), ((), ())),
        preferred_element_type=jnp.float32)
    out_ref[...] = s_blk.reshape(out_ref.shape)


def _softmax_body(sa_ref, sb_ref, dest_ref, pa_ref, pb_ref, logp_ref):
    n_per_w = _PER_W * _CHUNK
    dest = dest_ref[0]

    sa = sa_ref[...].reshape(sa_ref.shape[0], sa_ref.shape[2])
    fa = (lax.broadcasted_iota(jnp.int32, sa.shape, 0) * _TC_BLK
          + lax.broadcasted_iota(jnp.int32, sa.shape, 1))

    sb = sb_ref[...]
    rb = lax.broadcasted_iota(jnp.int32, sb.shape, 0)
    cb = lax.broadcasted_iota(jnp.int32, sb.shape, 1)
    fb = _TC_ROWS + rb * n_per_w + cb
    validb = (cb < n_per_w) & (fb < _N - 1)
    smb = jnp.where(validb, sb, jnp.float32(-1e30))

    m = jnp.maximum(jnp.max(sa), jnp.max(smb))
    ea = jnp.exp(sa - m)
    eb = jnp.where(validb, jnp.exp(smb - m), jnp.float32(0.0))
    tot = jnp.sum(ea) + jnp.sum(eb)
    pa_ref[...] = (ea / tot).reshape(pa_ref.shape)
    pb_ref[...] = eb / tot
    sd = (jnp.sum(jnp.where(fa == dest, sa, jnp.float32(0.0)))
          + jnp.sum(jnp.where(validb & (fb == dest), smb, jnp.float32(0.0))))
    logp_ref[...] = jnp.reshape(sd - m - jnp.log(tot), (1, 1))


def kernel(hv, W1, b1, W2, b2, bond_type, dest):
    n, d = hv.shape
    d_in = W1.shape[0]
    del b1, b2, bond_type

    w1af = W1[:d, :].reshape(-1)
    w2f = W2.reshape(-1)
    w2d = jnp.concatenate([w2f, w2f])
    sc_scores = _sc_scores(hv, w1af, w2d).reshape(_NW, _WSTRIDE)

    nsteps = _TC_ROWS // _TC_BLK
    w2r = W2.reshape(1, d_in)
    tc_scores = pl.pallas_call(
        _tc_scores_body,
        grid=(nsteps,),
        in_specs=[
            pl.BlockSpec((_TC_BLK, d), lambda i: (i, 0)),
            pl.BlockSpec((d_in, d_in), lambda i: (0, 0)),
            pl.BlockSpec((1, d_in), lambda i: (0, 0)),
        ],
        out_specs=pl.BlockSpec((1, 1, _TC_BLK), lambda i: (i, 0, 0)),
        out_shape=jax.ShapeDtypeStruct((nsteps, 1, _TC_BLK), jnp.float32),
        scratch_shapes=[pltpu.VMEM((1, d), jnp.float32)],
    )(hv, W1, w2r)

    dest_arr = jnp.asarray(dest, jnp.int32).reshape(1)
    pa, pb, logp = pl.pallas_call(
        _softmax_body,
        in_specs=[
            pl.BlockSpec((nsteps, 1, _TC_BLK), lambda: (0, 0, 0)),
            pl.BlockSpec((_NW, _WSTRIDE), lambda: (0, 0)),
            pl.BlockSpec(memory_space=pltpu.SMEM),
        ],
        out_specs=[
            pl.BlockSpec((nsteps, 1, _TC_BLK), lambda: (0, 0, 0)),
            pl.BlockSpec((_NW, _WSTRIDE), lambda: (0, 0)),
            pl.BlockSpec((1, 1), lambda: (0, 0)),
        ],
        out_shape=[
            jax.ShapeDtypeStruct((nsteps, 1, _TC_BLK), jnp.float32),
            jax.ShapeDtypeStruct((_NW, _WSTRIDE), jnp.float32),
            jax.ShapeDtypeStruct((1, 1), jnp.float32),
        ],
    )(tc_scores, sc_scores, dest_arr)

    probs = jnp.concatenate(
        [pa.reshape(1, _TC_ROWS),
         pb[:, : _PER_W * _CHUNK].reshape(1, _SC_ROWS)], axis=1)[:, : n - 1]
    return probs, logp


# speedup vs baseline: 1.0247x; 1.0247x over previous
import functools

import jax
import jax.numpy as jnp
from jax import lax
from jax.experimental import pallas as pl
from jax.experimental.pallas import tpu as pltpu
from jax.experimental.pallas import tpu_sc as plsc

_L = 16
_D = 128
_D_IN = 260
_N = 100000
_CHUNK = 125
_BUFROWS = 136
_NW = 32
_PER_W = 7
_SC_ROWS = _NW * _PER_W * _CHUNK
_TC_ROWS = _N - _SC_ROWS
_TC_BLK = 8000
_WSTRIDE = 880


def _sc_scores_body(hv_hbm, w1a_hbm, w2d_hbm, out_hbm,
                    buf0, buf1, buf2, w1buf, w2buf, va2buf, sbuf,
                    semw, sem0, sem1, sem2):
    wid = lax.axis_index("s") * 2 + lax.axis_index("c")
    iota = lax.iota(jnp.int32, _L)
    first = wid * _PER_W

    def chunk_src(t):
        start = _TC_ROWS + (first + t) * _CHUNK
        aligned = pl.multiple_of(
            jnp.minimum((start // 8) * 8, _N - _BUFROWS), 8)
        return hv_hbm.at[pl.ds(aligned, _BUFROWS)], start - aligned

    def fire(t, buf, sem):
        src, _ = chunk_src(t)
        pltpu.async_copy(src, buf, sem)

    pltpu.async_copy(w1a_hbm, w1buf, semw)
    fire(0, buf0, sem0)
    pltpu.sync_copy(w2d_hbm, w2buf)
    pltpu.make_async_copy(w1a_hbm, w1buf, semw).wait()

    w1base = [(16 * j + iota) * _D_IN for j in range(_D // _L)]

    def va_step(k, accs):
        kv = k + iota
        kv = jnp.where(kv >= _D_IN, kv - _D_IN, kv)
        w2k = w2buf[pl.ds(k, _L)]
        return tuple(
            accs[j] + plsc.load_gather(w1buf, [w1base[j] + kv]) * w2k
            for j in range(_D // _L))

    va = lax.fori_loop(0, _D_IN, va_step,
                       tuple(jnp.zeros((_L,), jnp.float32)
                             for _ in range(_D // _L)), unroll=4)
    for j in range(_D // _L):
        va2buf[pl.ds(16 * j, _L)] = va[j]
        va2buf[pl.ds(_D + 16 * j, _L)] = va[j]

    ngroups = (_CHUNK + _L - 1) // _L

    def compute(t, buf):
        _, extra = chunk_src(t)
        rows = [jnp.minimum(16 * g + iota, _CHUNK - 1) + extra
                for g in range(ngroups)]

        def d_step(d, accs):
            m = jnp.bitwise_and(d + iota, _D - 1)
            vad = va2buf[pl.ds(d, _L)]
            return tuple(
                accs[g] + plsc.load_gather(buf, [rows[g], m]) * vad
                for g in range(ngroups))

        accs = lax.fori_loop(
            0, _D, d_step,
            tuple(jnp.zeros((_L,), jnp.float32) for _ in range(ngroups)),
            unroll=4)
        for g in range(ngroups):
            sbuf[pl.ds(t * _CHUNK + 16 * g, _L)] = accs[g]

    def wait(t, buf, sem):
        src, _ = chunk_src(t)
        pltpu.make_async_copy(src, buf, sem).wait()

    fire(1, buf1, sem1)
    fire(2, buf2, sem2)

    def triple_step(p, carry):
        t0 = 3 * p
        for i, (b, sm) in enumerate(((buf0, sem0), (buf1, sem1),
                                     (buf2, sem2))):
            t = t0 + i
            wait(t, b, sm)
            compute(t, b)

            @pl.when(t + 3 < _PER_W)
            def _():
                fire(t + 3, b, sm)
        return carry

    lax.fori_loop(0, _PER_W // 3, triple_step, 0)
    wait(_PER_W - 1, buf0, sem0)
    compute(_PER_W - 1, buf0)
    pltpu.sync_copy(sbuf, out_hbm.at[pl.ds(wid * _WSTRIDE, _WSTRIDE)])


def _sc_scores(hv, w1af, w2d):
    mesh = plsc.VectorSubcoreMesh(core_axis_name="c", subcore_axis_name="s")
    f = functools.partial(
        pl.kernel, mesh=mesh,
        out_type=jax.ShapeDtypeStruct((_NW * _WSTRIDE,), jnp.float32),
        scratch_types=[
            pltpu.VMEM((_BUFROWS, _D), jnp.float32),
            pltpu.VMEM((_BUFROWS, _D), jnp.float32),
            pltpu.VMEM((_BUFROWS, _D), jnp.float32),
            pltpu.VMEM((_D * _D_IN,), jnp.float32),
            pltpu.VMEM((2 * _D_IN,), jnp.float32),
            pltpu.VMEM((2 * _D,), jnp.float32),
            pltpu.VMEM((_WSTRIDE,), jnp.float32),
            pltpu.SemaphoreType.DMA,
            pltpu.SemaphoreType.DMA,
            pltpu.SemaphoreType.DMA,
            pltpu.SemaphoreType.DMA,
        ],
        compiler_params=pltpu.CompilerParams(needs_layout_passes=False),
    )(_sc_scores_body)
    return f(hv, w1af, w2d)


def _tc_scores_body(hv_ref, w1_ref, w2r_ref, out_ref, va_ref):
    d = hv_ref.shape[1]

    @pl.when(pl.program_id(0) == 0)
    def _():
        va_ref[...] = lax.dot_general(
            w2r_ref[...], w1_ref[0:d, :], (((1,), (1,)), ((), ())),
            preferred_element_type=jnp.float32)

    s_blk = lax.dot_general(
        va_ref[...], hv_ref[...], (((1,), (1,)), ((), ())),
        preferred_element_type=jnp.float32)
    out_ref[...] = s_blk.reshape(out_ref.shape)


def _softmax_body(sa_ref, sb_ref, dest_ref, pa_ref, pb_ref, logp_ref):
    n_per_w = _PER_W * _CHUNK
    dest = dest_ref[0]

    sa = sa_ref[...].reshape(sa_ref.shape[0], sa_ref.shape[2])
    fa = (lax.broadcasted_iota(jnp.int32, sa.shape, 0) * _TC_BLK
          + lax.broadcasted_iota(jnp.int32, sa.shape, 1))

    sb = sb_ref[...]
    rb = lax.broadcasted_iota(jnp.int32, sb.shape, 0)
    cb = lax.broadcasted_iota(jnp.int32, sb.shape, 1)
    fb = _TC_ROWS + rb * n_per_w + cb
    validb = (cb < n_per_w) & (fb < _N - 1)
    smb = jnp.where(validb, sb, jnp.float32(-1e30))

    m = jnp.maximum(jnp.max(sa), jnp.max(smb))
    ea = jnp.exp(sa - m)
    eb = jnp.where(validb, jnp.exp(smb - m), jnp.float32(0.0))
    tot = jnp.sum(ea) + jnp.sum(eb)
    pa_ref[...] = (ea / tot).reshape(pa_ref.shape)
    pb_ref[...] = eb / tot
    sd = (jnp.sum(jnp.where(fa == dest, sa, jnp.float32(0.0)))
          + jnp.sum(jnp.where(validb & (fb == dest), smb, jnp.float32(0.0))))
    logp_ref[...] = jnp.reshape(sd - m - jnp.log(tot), (1, 1))


def kernel(hv, W1, b1, W2, b2, bond_type, dest):
    n, d = hv.shape
    d_in = W1.shape[0]
    del b1, b2, bond_type

    nsteps = _TC_ROWS // _TC_BLK
    w2r = W2.reshape(1, d_in)
    tc_scores = pl.pallas_call(
        _tc_scores_body,
        grid=(nsteps,),
        in_specs=[
            pl.BlockSpec((_TC_BLK, d), lambda i: (i, 0)),
            pl.BlockSpec((d_in, d_in), lambda i: (0, 0)),
            pl.BlockSpec((1, d_in), lambda i: (0, 0)),
        ],
        out_specs=pl.BlockSpec((1, 1, _TC_BLK), lambda i: (i, 0, 0)),
        out_shape=jax.ShapeDtypeStruct((nsteps, 1, _TC_BLK), jnp.float32),
        scratch_shapes=[pltpu.VMEM((1, d), jnp.float32)],
    )(hv, W1, w2r)

    w1af = W1[:d, :].reshape(-1)
    w2f = W2.reshape(-1)
    w2d = jnp.concatenate([w2f, w2f])
    sc_scores = _sc_scores(hv, w1af, w2d).reshape(_NW, _WSTRIDE)

    dest_arr = jnp.asarray(dest, jnp.int32).reshape(1)
    pa, pb, logp = pl.pallas_call(
        _softmax_body,
        in_specs=[
            pl.BlockSpec((nsteps, 1, _TC_BLK), lambda: (0, 0, 0)),
            pl.BlockSpec((_NW, _WSTRIDE), lambda: (0, 0)),
            pl.BlockSpec(memory_space=pltpu.SMEM),
        ],
        out_specs=[
            pl.BlockSpec((nsteps, 1, _TC_BLK), lambda: (0, 0, 0)),
            pl.BlockSpec((_NW, _WSTRIDE), lambda: (0, 0)),
            pl.BlockSpec((1, 1), lambda: (0, 0)),
        ],
        out_shape=[
            jax.ShapeDtypeStruct((nsteps, 1, _TC_BLK), jnp.float32),
            jax.ShapeDtypeStruct((_NW, _WSTRIDE), jnp.float32),
            jax.ShapeDtypeStruct((1, 1), jnp.float32),
        ],
    )(tc_scores, sc_scores, dest_arr)

    probs = jnp.concatenate(
        [pa.reshape(1, _TC_ROWS),
         pb[:, : _PER_W * _CHUNK].reshape(1, _SC_ROWS)], axis=1)[:, : n - 1]
    return probs, logp
